# baseline (speedup 1.0000x reference)
import functools

import jax
import jax.numpy as jnp
from jax import lax
from jax.experimental import pallas as pl
from jax.experimental.pallas import tpu as pltpu
from jax.experimental.pallas import tpu_sc as plsc

N_USERS = 10000
N_ITEMS = 10000
D = 128
N_LAYERS = 2

NC, NS, L = 2, 16, 16
NW = NC * NS
CHUNK = 112
NBUF = 3
EDGE_ALIGN = NW * CHUNK * NBUF


SPLIT_NUM, SPLIT_DEN = 7, 8


def _chunk_split(ep):
    t_pair = ep // (NS * CHUNK)
    c0 = (t_pair * SPLIT_NUM // SPLIT_DEN) // NBUF * NBUF
    c1 = t_pair - c0
    assert c0 % NBUF == 0 and c1 % NBUF == 0 and c0 >= 2 * NBUF and c1 >= 2 * NBUF
    return c0, c1


def _pad_edges(idx_a, idx_b, vals):
    e = idx_a.shape[0]
    ep = ((e + EDGE_ALIGN - 1) // EDGE_ALIGN) * EDGE_ALIGN
    pad = ep - e
    idx_a = jnp.pad(idx_a, (0, pad)).reshape(ep // CHUNK, CHUNK)
    idx_b = jnp.pad(idx_b, (0, pad)).reshape(ep // CHUNK, CHUNK)
    vals = jnp.pad(vals, (0, pad)).reshape(ep // CHUNK, CHUNK)
    return idx_a, idx_b, vals, ep


N_PAD = 10240


@functools.partial(jax.jit, static_argnames=("ep", "n_dst"))
def _seg_sum_partials(src_emb, idx_src, idx_dst, vals, zeros, *, ep, n_dst):
    c0_chunks, c1_chunks = _chunk_split(ep)
    rows_per_tile = n_dst // NS

    def body(src_hbm, isrc_hbm, idst_hbm, vals_hbm, zeros_hbm, out_hbm,
             isrc0, isrc1, isrc2, idst0, idst1, idst2, valb0, valb1, valb2,
             rows0, rows1, rows2, acc,
             sg0, sg1, sg2, ss0, ss1, ss2, si0, si1, si2, sd0, sd1, sd2):
        isrc = (isrc0, isrc1, isrc2)
        idst = (idst0, idst1, idst2)
        valb = (valb0, valb1, valb2)
        rows = (rows0, rows1, rows2)
        sg = (sg0, sg1, sg2)
        ss = (ss0, ss1, ss2)
        si = (si0, si1, si2)
        sd = (sd0, sd1, sd2)
        cid = lax.axis_index("c")
        sid = lax.axis_index("s")
        r0 = sid * rows_per_tile
        cn = jnp.where(cid == 0, c0_chunks, c1_chunks)
        base = jnp.where(cid == 0, sid * c0_chunks,
                         NS * c0_chunks + sid * c1_chunks)

        def copy_iv(j, b):
            pltpu.async_copy(isrc_hbm.at[base + j], isrc[b], si[b])
            pltpu.async_copy(vals_hbm.at[base + j], valb[b], si[b])

        def wait_iv(j, b):
            pltpu.make_async_copy(isrc_hbm.at[base + j], isrc[b],
                                  si[b]).wait()
            pltpu.make_async_copy(vals_hbm.at[base + j], valb[b],
                                  si[b]).wait()

        def copy_id(j, b):
            pltpu.async_copy(idst_hbm.at[base + j], idst[b], sd[b])

        def wait_id(j, b):
            pltpu.make_async_copy(idst_hbm.at[base + j], idst[b], sd[b]).wait()

        def gather(j, b):
            pltpu.async_copy(src_hbm.at[isrc[b]], rows[b], sg[b])

        def gather_wait(j, b):
            pltpu.make_async_copy(src_hbm.at[isrc[b]], rows[b], sg[b]).wait()

        def scatter(j, b):
            pltpu.async_copy(rows[b], acc.at[idst[b]], ss[b], add=True)

        def scatter_wait(j, b):
            pltpu.make_async_copy(rows[b], acc.at[idst[b]], ss[b]).wait()

        def scale(b):
            rv = rows[b]
            vb = valb[b]

            def mul_body(k, c2):
                vv = vb[pl.ds(k * L, L)]
                for t in range(L):
                    v = vv[t]
                    e = k * L + t
                    for g in range(D // L):
                        sl = pl.ds(g * L, L)
                        rv[e, sl] = rv[e, sl] * v
                return c2

            lax.fori_loop(0, CHUNK // L, mul_body, 0)

        pltpu.sync_copy(isrc_hbm.at[base], isrc[0])
        pltpu.sync_copy(vals_hbm.at[base], valb[0])
        pltpu.sync_copy(isrc_hbm.at[base + 1], isrc[1])
        pltpu.sync_copy(vals_hbm.at[base + 1], valb[1])
        gather(0, 0)
        gather(1, 1)
        copy_iv(2, 2)
        copy_id(0, 0)
        copy_id(1, 1)

        pltpu.sync_copy(zeros_hbm.at[pl.ds(r0, rows_per_tile)],
                        acc.at[pl.ds(r0, rows_per_tile)])
        plsc.subcore_barrier()

        def outer(jj, carry):
            j0 = jj * NBUF
            for p in range(NBUF):
                j = j0 + p
                b2 = (p + 2) % NBUF
                gather_wait(j, p)
                scale(p)
                wait_id(j, p)
                scatter(j, p)

                @pl.when(j0 + p + 3 < cn)
                def _():
                    copy_iv(j + 3, p)

                if p == 0:
                    @pl.when(jj > 0)
                    def _():
                        scatter_wait(j - 1, b2)
                    copy_id(j + 2, b2)
                    wait_iv(j + 2, b2)
                    gather(j + 2, b2)
                else:
                    scatter_wait(j - 1, b2)

                    @pl.when(j0 + p + 2 < cn)
                    def _():
                        copy_id(j + 2, b2)
                        wait_iv(j + 2, b2)
                        gather(j + 2, b2)
            return carry

        lax.fori_loop(0, cn // NBUF, outer, 0)
        scatter_wait(cn - 1, NBUF - 1)

        plsc.subcore_barrier()
        pltpu.sync_copy(acc.at[pl.ds(r0, rows_per_tile)],
                        out_hbm.at[cid, pl.ds(r0, rows_per_tile)])

    mesh = plsc.VectorSubcoreMesh(core_axis_name="c", subcore_axis_name="s")
    iv = pltpu.VMEM((CHUNK,), jnp.int32)
    fv = pltpu.VMEM((CHUNK,), jnp.float32)
    rv = pltpu.VMEM((CHUNK, D), jnp.float32)
    sem = pltpu.SemaphoreType.DMA
    fn = pl.kernel(
        body,
        out_type=jax.ShapeDtypeStruct((NC, n_dst, D), jnp.float32),
        mesh=mesh,
        scratch_types=(
            [iv] * 3 + [iv] * 3 + [fv] * 3 + [rv] * 3
            + [pltpu.VMEM_SHARED((n_dst, D), jnp.float32)]
            + [sem] * 12
        ),
    )
    return fn(src_emb, idx_src, idx_dst, vals, zeros)


def _tc_layer(pu, pit, ps, cur_u, cur_i, cur_s, W_ui_i, b_ui_i, W_s_i, b_s_i,
              g0, bb0, g1, bb1):
    R = 1000
    grid = N_USERS // R
    eps = 1e-5

    def ln(x, g, b):
        m = jnp.mean(x, axis=-1, keepdims=True)
        xc = x - m
        v = jnp.mean(xc * xc, axis=-1, keepdims=True)
        return xc * lax.rsqrt(v + eps) * g + b

    def body(pu0, pu1, pit0, pit1, ps0, ps1, cu, ci, cs,
             wui, bui, ws, bs, g0r, b0r, g1r, b1r, ou, oi, osr):
        dn = (((1,), (1,)), ((), ()))
        au = pu0[...] + pu1[...]
        hu = lax.dot_general(au, wui[...], dn,
                             preferred_element_type=jnp.float32) + bui[...]
        ou[...] = ln(cu[...] + hu, g0r[...], b0r[...])
        ai = pit0[...] + pit1[...]
        hi = lax.dot_general(ai, wui[...], dn,
                             preferred_element_type=jnp.float32) + bui[...]
        oi[...] = ln(ci[...] + hi, g1r[...], b1r[...])
        as_ = ps0[...] + ps1[...]
        hs = lax.dot_general(as_, ws[...], dn,
                             preferred_element_type=jnp.float32) + bs[...]
        osr[...] = ln(cs[...] + hs, g0r[...], b0r[...])

    blk = pl.BlockSpec((R, D), lambda i: (i, 0))
    wblk = pl.BlockSpec((D, D), lambda i: (0, 0))
    vblk = pl.BlockSpec((1, D), lambda i: (0, 0))
    out_shape = jax.ShapeDtypeStruct((N_USERS, D), jnp.float32)
    return pl.pallas_call(
        body,
        grid=(grid,),
        in_specs=[blk] * 9 + [wblk, vblk, wblk, vblk, vblk, vblk, vblk, vblk],
        out_specs=[blk, blk, blk],
        out_shape=[out_shape, out_shape, out_shape],
    )(pu[0], pu[1], pit[0], pit[1], ps[0], ps[1], cur_u, cur_i, cur_s,
      W_ui_i, b_ui_i.reshape(1, D), W_s_i, b_s_i.reshape(1, D),
      g0.reshape(1, D), bb0.reshape(1, D), g1.reshape(1, D), bb1.reshape(1, D))


def kernel(user_emb, item_emb, ui_values, social_values, W_ui, b_ui, W_s, b_s,
           ln_g, ln_b, ui_edge_index, social_edge_index):
    row = ui_edge_index[0].astype(jnp.int32)
    col = ui_edge_index[1].astype(jnp.int32)
    srow = social_edge_index[0].astype(jnp.int32)
    scol = social_edge_index[1].astype(jnp.int32)

    row_p, col_p, uvals_p, e_ui = _pad_edges(row, col, ui_values)
    srow_p, scol_p, svals_p, e_soc = _pad_edges(srow, scol, social_values)

    zeros = jnp.zeros((N_PAD, D), jnp.float32)

    cur_u, cur_i, cur_s = user_emb, item_emb, user_emb
    ui_list = [user_emb]
    s_list = [user_emb]
    for i in range(N_LAYERS):
        pu = _seg_sum_partials(cur_i, col_p, row_p, uvals_p, zeros,
                               ep=e_ui, n_dst=N_PAD)
        pit = _seg_sum_partials(cur_u, row_p, col_p, uvals_p, zeros,
                                ep=e_ui, n_dst=N_PAD)
        ps = _seg_sum_partials(cur_s, scol_p, srow_p, svals_p, zeros,
                               ep=e_soc, n_dst=N_PAD)
        cur_u, cur_i, cur_s = _tc_layer(
            pu, pit, ps, cur_u, cur_i, cur_s,
            W_ui[i], b_ui[i], W_s[i], b_s[i],
            ln_g[2 * i], ln_b[2 * i], ln_g[2 * i + 1], ln_b[2 * i + 1])
        ui_list.append(cur_u)
        s_list.append(cur_s)

    return jnp.concatenate([jnp.stack(ui_list), jnp.stack(s_list),
                            cur_i[None]], axis=0)


# speedup vs baseline: 1.0287x; 1.0287x over previous
import functools

import jax
import jax.numpy as jnp
from jax import lax
from jax.experimental import pallas as pl
from jax.experimental.pallas import tpu as pltpu
from jax.experimental.pallas import tpu_sc as plsc

N_USERS = 10000
N_ITEMS = 10000
D = 128
N_LAYERS = 2

NC, NS, L = 2, 16, 16
NW = NC * NS
CHUNK = 112
NBUF = 3
EDGE_ALIGN = NW * CHUNK * NBUF


SPLIT_NUM, SPLIT_DEN = 5, 6


def _chunk_split(ep):
    t_pair = ep // (NS * CHUNK)
    c0 = (t_pair * SPLIT_NUM // SPLIT_DEN) // NBUF * NBUF
    c1 = t_pair - c0
    assert c0 % NBUF == 0 and c1 % NBUF == 0 and c0 >= 2 * NBUF and c1 >= 2 * NBUF
    return c0, c1


def _pad_edges(idx_a, idx_b, vals):
    e = idx_a.shape[0]
    ep = ((e + EDGE_ALIGN - 1) // EDGE_ALIGN) * EDGE_ALIGN
    pad = ep - e
    idx_a = jnp.pad(idx_a, (0, pad)).reshape(ep // CHUNK, CHUNK)
    idx_b = jnp.pad(idx_b, (0, pad)).reshape(ep // CHUNK, CHUNK)
    vals = jnp.pad(vals, (0, pad)).reshape(ep // CHUNK, CHUNK)
    return idx_a, idx_b, vals, ep


N_PAD = 10240


@functools.partial(jax.jit, static_argnames=("ep", "n_dst"))
def _seg_sum_partials(src_emb, idx_src, idx_dst, vals, zeros, *, ep, n_dst):
    c0_chunks, c1_chunks = _chunk_split(ep)
    rows_per_tile = n_dst // NS

    def body(src_hbm, isrc_hbm, idst_hbm, vals_hbm, zeros_hbm, out_hbm,
             isrc0, isrc1, isrc2, idst0, idst1, idst2, valb0, valb1, valb2,
             rows0, rows1, rows2, acc,
             sg0, sg1, sg2, ss0, ss1, ss2, si0, si1, si2, sd0, sd1, sd2):
        isrc = (isrc0, isrc1, isrc2)
        idst = (idst0, idst1, idst2)
        valb = (valb0, valb1, valb2)
        rows = (rows0, rows1, rows2)
        sg = (sg0, sg1, sg2)
        ss = (ss0, ss1, ss2)
        si = (si0, si1, si2)
        sd = (sd0, sd1, sd2)
        cid = lax.axis_index("c")
        sid = lax.axis_index("s")
        r0 = sid * rows_per_tile
        cn = jnp.where(cid == 0, c0_chunks, c1_chunks)
        base = jnp.where(cid == 0, sid * c0_chunks,
                         NS * c0_chunks + sid * c1_chunks)

        def copy_iv(j, b):
            pltpu.async_copy(isrc_hbm.at[base + j], isrc[b], si[b])
            pltpu.async_copy(vals_hbm.at[base + j], valb[b], si[b])

        def wait_iv(j, b):
            pltpu.make_async_copy(isrc_hbm.at[base + j], isrc[b],
                                  si[b]).wait()
            pltpu.make_async_copy(vals_hbm.at[base + j], valb[b],
                                  si[b]).wait()

        def copy_id(j, b):
            pltpu.async_copy(idst_hbm.at[base + j], idst[b], sd[b])

        def wait_id(j, b):
            pltpu.make_async_copy(idst_hbm.at[base + j], idst[b], sd[b]).wait()

        def gather(j, b):
            pltpu.async_copy(src_hbm.at[isrc[b]], rows[b], sg[b])

        def gather_wait(j, b):
            pltpu.make_async_copy(src_hbm.at[isrc[b]], rows[b], sg[b]).wait()

        def scatter(j, b):
            pltpu.async_copy(rows[b], acc.at[idst[b]], ss[b], add=True)

        def scatter_wait(j, b):
            pltpu.make_async_copy(rows[b], acc.at[idst[b]], ss[b]).wait()

        def scale(b):
            rv = rows[b]
            vb = valb[b]

            def mul_body(k, c2):
                vv = vb[pl.ds(k * L, L)]
                for t in range(L):
                    v = vv[t]
                    e = k * L + t
                    for g in range(D // L):
                        sl = pl.ds(g * L, L)
                        rv[e, sl] = rv[e, sl] * v
                return c2

            lax.fori_loop(0, CHUNK // L, mul_body, 0)

        pltpu.sync_copy(isrc_hbm.at[base], isrc[0])
        pltpu.sync_copy(vals_hbm.at[base], valb[0])
        pltpu.sync_copy(isrc_hbm.at[base + 1], isrc[1])
        pltpu.sync_copy(vals_hbm.at[base + 1], valb[1])
        gather(0, 0)
        gather(1, 1)
        copy_iv(2, 2)
        copy_id(0, 0)
        copy_id(1, 1)

        pltpu.sync_copy(zeros_hbm.at[pl.ds(r0, rows_per_tile)],
                        acc.at[pl.ds(r0, rows_per_tile)])
        plsc.subcore_barrier()

        def outer(jj, carry):
            j0 = jj * NBUF
            for p in range(NBUF):
                j = j0 + p
                b2 = (p + 2) % NBUF
                gather_wait(j, p)
                scale(p)
                wait_id(j, p)
                scatter(j, p)

                @pl.when(j0 + p + 3 < cn)
                def _():
                    copy_iv(j + 3, p)

                if p == 0:
                    @pl.when(jj > 0)
                    def _():
                        scatter_wait(j - 1, b2)
                    copy_id(j + 2, b2)
                    wait_iv(j + 2, b2)
                    gather(j + 2, b2)
                else:
                    scatter_wait(j - 1, b2)

                    @pl.when(j0 + p + 2 < cn)
                    def _():
                        copy_id(j + 2, b2)
                        wait_iv(j + 2, b2)
                        gather(j + 2, b2)
            return carry

        lax.fori_loop(0, cn // NBUF, outer, 0)
        scatter_wait(cn - 1, NBUF - 1)

        plsc.subcore_barrier()
        pltpu.sync_copy(acc.at[pl.ds(r0, rows_per_tile)],
                        out_hbm.at[cid, pl.ds(r0, rows_per_tile)])

    mesh = plsc.VectorSubcoreMesh(core_axis_name="c", subcore_axis_name="s")
    iv = pltpu.VMEM((CHUNK,), jnp.int32)
    fv = pltpu.VMEM((CHUNK,), jnp.float32)
    rv = pltpu.VMEM((CHUNK, D), jnp.float32)
    sem = pltpu.SemaphoreType.DMA
    fn = pl.kernel(
        body,
        out_type=jax.ShapeDtypeStruct((NC, n_dst, D), jnp.float32),
        mesh=mesh,
        scratch_types=(
            [iv] * 3 + [iv] * 3 + [fv] * 3 + [rv] * 3
            + [pltpu.VMEM_SHARED((n_dst, D), jnp.float32)]
            + [sem] * 12
        ),
    )
    return fn(src_emb, idx_src, idx_dst, vals, zeros)


def _tc_layer(pu, pit, ps, cur_u, cur_i, cur_s, W_ui_i, b_ui_i, W_s_i, b_s_i,
              g0, bb0, g1, bb1):
    R = 1000
    grid = N_USERS // R
    eps = 1e-5

    def ln(x, g, b):
        m = jnp.mean(x, axis=-1, keepdims=True)
        xc = x - m
        v = jnp.mean(xc * xc, axis=-1, keepdims=True)
        return xc * lax.rsqrt(v + eps) * g + b

    def body(pu0, pu1, pit0, pit1, ps0, ps1, cu, ci, cs,
             wui, bui, ws, bs, g0r, b0r, g1r, b1r, ou, oi, osr):
        dn = (((1,), (1,)), ((), ()))
        au = pu0[...] + pu1[...]
        hu = lax.dot_general(au, wui[...], dn,
                             preferred_element_type=jnp.float32) + bui[...]
        ou[...] = ln(cu[...] + hu, g0r[...], b0r[...])
        ai = pit0[...] + pit1[...]
        hi = lax.dot_general(ai, wui[...], dn,
                             preferred_element_type=jnp.float32) + bui[...]
        oi[...] = ln(ci[...] + hi, g1r[...], b1r[...])
        as_ = ps0[...] + ps1[...]
        hs = lax.dot_general(as_, ws[...], dn,
                             preferred_element_type=jnp.float32) + bs[...]
        osr[...] = ln(cs[...] + hs, g0r[...], b0r[...])

    blk = pl.BlockSpec((R, D), lambda i: (i, 0))
    wblk = pl.BlockSpec((D, D), lambda i: (0, 0))
    vblk = pl.BlockSpec((1, D), lambda i: (0, 0))
    out_shape = jax.ShapeDtypeStruct((N_USERS, D), jnp.float32)
    return pl.pallas_call(
        body,
        grid=(grid,),
        in_specs=[blk] * 9 + [wblk, vblk, wblk, vblk, vblk, vblk, vblk, vblk],
        out_specs=[blk, blk, blk],
        out_shape=[out_shape, out_shape, out_shape],
    )(pu[0], pu[1], pit[0], pit[1], ps[0], ps[1], cur_u, cur_i, cur_s,
      W_ui_i, b_ui_i.reshape(1, D), W_s_i, b_s_i.reshape(1, D),
      g0.reshape(1, D), bb0.reshape(1, D), g1.reshape(1, D), bb1.reshape(1, D))


def kernel(user_emb, item_emb, ui_values, social_values, W_ui, b_ui, W_s, b_s,
           ln_g, ln_b, ui_edge_index, social_edge_index):
    row = ui_edge_index[0].astype(jnp.int32)
    col = ui_edge_index[1].astype(jnp.int32)
    srow = social_edge_index[0].astype(jnp.int32)
    scol = social_edge_index[1].astype(jnp.int32)

    row_p, col_p, uvals_p, e_ui = _pad_edges(row, col, ui_values)
    srow_p, scol_p, svals_p, e_soc = _pad_edges(srow, scol, social_values)

    zeros = jnp.zeros((N_PAD, D), jnp.float32)

    cur_u, cur_i, cur_s = user_emb, item_emb, user_emb
    ui_list = [user_emb]
    s_list = [user_emb]
    for i in range(N_LAYERS):
        pu = _seg_sum_partials(cur_i, col_p, row_p, uvals_p, zeros,
                               ep=e_ui, n_dst=N_PAD)
        pit = _seg_sum_partials(cur_u, row_p, col_p, uvals_p, zeros,
                                ep=e_ui, n_dst=N_PAD)
        ps = _seg_sum_partials(cur_s, scol_p, srow_p, svals_p, zeros,
                               ep=e_soc, n_dst=N_PAD)
        cur_u, cur_i, cur_s = _tc_layer(
            pu, pit, ps, cur_u, cur_i, cur_s,
            W_ui[i], b_ui[i], W_s[i], b_s[i],
            ln_g[2 * i], ln_b[2 * i], ln_g[2 * i + 1], ln_b[2 * i + 1])
        ui_list.append(cur_u)
        s_list.append(cur_s)

    return jnp.concatenate([jnp.stack(ui_list), jnp.stack(s_list),
                            cur_i[None]], axis=0)


# speedup vs baseline: 1.0325x; 1.0037x over previous
import functools

import jax
import jax.numpy as jnp
from jax import lax
from jax.experimental import pallas as pl
from jax.experimental.pallas import tpu as pltpu
from jax.experimental.pallas import tpu_sc as plsc

N_USERS = 10000
N_ITEMS = 10000
D = 128
N_LAYERS = 2

NC, NS, L = 2, 16, 16
NW = NC * NS
CHUNK = 112
NBUF = 3
EDGE_ALIGN = NW * CHUNK * NBUF


SPLIT_NUM, SPLIT_DEN = 6, 7


def _chunk_split(ep):
    t_pair = ep // (NS * CHUNK)
    c0 = (t_pair * SPLIT_NUM // SPLIT_DEN) // NBUF * NBUF
    c1 = t_pair - c0
    assert c0 % NBUF == 0 and c1 % NBUF == 0 and c0 >= 2 * NBUF and c1 >= 2 * NBUF
    return c0, c1


def _pad_edges(idx_a, idx_b, vals):
    e = idx_a.shape[0]
    ep = ((e + EDGE_ALIGN - 1) // EDGE_ALIGN) * EDGE_ALIGN
    pad = ep - e
    idx_a = jnp.pad(idx_a, (0, pad)).reshape(ep // CHUNK, CHUNK)
    idx_b = jnp.pad(idx_b, (0, pad)).reshape(ep // CHUNK, CHUNK)
    vals = jnp.pad(vals, (0, pad)).reshape(ep // CHUNK, CHUNK)
    return idx_a, idx_b, vals, ep


N_PAD = 10240


@functools.partial(jax.jit, static_argnames=("ep", "n_dst"))
def _seg_sum_partials(src_emb, idx_src, idx_dst, vals, zeros, *, ep, n_dst):
    c0_chunks, c1_chunks = _chunk_split(ep)
    rows_per_tile = n_dst // NS

    def body(src_hbm, isrc_hbm, idst_hbm, vals_hbm, zeros_hbm, out_hbm,
             isrc0, isrc1, isrc2, idst0, idst1, idst2, valb0, valb1, valb2,
             rows0, rows1, rows2, acc,
             sg0, sg1, sg2, ss0, ss1, ss2, si0, si1, si2, sd0, sd1, sd2):
        isrc = (isrc0, isrc1, isrc2)
        idst = (idst0, idst1, idst2)
        valb = (valb0, valb1, valb2)
        rows = (rows0, rows1, rows2)
        sg = (sg0, sg1, sg2)
        ss = (ss0, ss1, ss2)
        si = (si0, si1, si2)
        sd = (sd0, sd1, sd2)
        cid = lax.axis_index("c")
        sid = lax.axis_index("s")
        r0 = sid * rows_per_tile
        cn = jnp.where(cid == 0, c0_chunks, c1_chunks)
        base = jnp.where(cid == 0, sid * c0_chunks,
                         NS * c0_chunks + sid * c1_chunks)

        def copy_iv(j, b):
            pltpu.async_copy(isrc_hbm.at[base + j], isrc[b], si[b])
            pltpu.async_copy(vals_hbm.at[base + j], valb[b], si[b])

        def wait_iv(j, b):
            pltpu.make_async_copy(isrc_hbm.at[base + j], isrc[b],
                                  si[b]).wait()
            pltpu.make_async_copy(vals_hbm.at[base + j], valb[b],
                                  si[b]).wait()

        def copy_id(j, b):
            pltpu.async_copy(idst_hbm.at[base + j], idst[b], sd[b])

        def wait_id(j, b):
            pltpu.make_async_copy(idst_hbm.at[base + j], idst[b], sd[b]).wait()

        def gather(j, b):
            pltpu.async_copy(src_hbm.at[isrc[b]], rows[b], sg[b])

        def gather_wait(j, b):
            pltpu.make_async_copy(src_hbm.at[isrc[b]], rows[b], sg[b]).wait()

        def scatter(j, b):
            pltpu.async_copy(rows[b], acc.at[idst[b]], ss[b], add=True)

        def scatter_wait(j, b):
            pltpu.make_async_copy(rows[b], acc.at[idst[b]], ss[b]).wait()

        def scale(b):
            rv = rows[b]
            vb = valb[b]

            def mul_body(k, c2):
                vv = vb[pl.ds(k * L, L)]
                for t in range(L):
                    v = vv[t]
                    e = k * L + t
                    for g in range(D // L):
                        sl = pl.ds(g * L, L)
                        rv[e, sl] = rv[e, sl] * v
                return c2

            lax.fori_loop(0, CHUNK // L, mul_body, 0)

        pltpu.sync_copy(isrc_hbm.at[base], isrc[0])
        pltpu.sync_copy(vals_hbm.at[base], valb[0])
        pltpu.sync_copy(isrc_hbm.at[base + 1], isrc[1])
        pltpu.sync_copy(vals_hbm.at[base + 1], valb[1])
        gather(0, 0)
        gather(1, 1)
        copy_iv(2, 2)
        copy_id(0, 0)
        copy_id(1, 1)

        pltpu.sync_copy(zeros_hbm.at[pl.ds(r0, rows_per_tile)],
                        acc.at[pl.ds(r0, rows_per_tile)])
        plsc.subcore_barrier()

        def outer(jj, carry):
            j0 = jj * NBUF
            for p in range(NBUF):
                j = j0 + p
                b2 = (p + 2) % NBUF
                gather_wait(j, p)
                scale(p)
                wait_id(j, p)
                scatter(j, p)

                @pl.when(j0 + p + 3 < cn)
                def _():
                    copy_iv(j + 3, p)

                if p == 0:
                    @pl.when(jj > 0)
                    def _():
                        scatter_wait(j - 1, b2)
                    copy_id(j + 2, b2)
                    wait_iv(j + 2, b2)
                    gather(j + 2, b2)
                else:
                    scatter_wait(j - 1, b2)

                    @pl.when(j0 + p + 2 < cn)
                    def _():
                        copy_id(j + 2, b2)
                        wait_iv(j + 2, b2)
                        gather(j + 2, b2)
            return carry

        lax.fori_loop(0, cn // NBUF, outer, 0)
        scatter_wait(cn - 1, NBUF - 1)

        plsc.subcore_barrier()
        pltpu.sync_copy(acc.at[pl.ds(r0, rows_per_tile)],
                        out_hbm.at[cid, pl.ds(r0, rows_per_tile)])

    mesh = plsc.VectorSubcoreMesh(core_axis_name="c", subcore_axis_name="s")
    iv = pltpu.VMEM((CHUNK,), jnp.int32)
    fv = pltpu.VMEM((CHUNK,), jnp.float32)
    rv = pltpu.VMEM((CHUNK, D), jnp.float32)
    sem = pltpu.SemaphoreType.DMA
    fn = pl.kernel(
        body,
        out_type=jax.ShapeDtypeStruct((NC, n_dst, D), jnp.float32),
        mesh=mesh,
        scratch_types=(
            [iv] * 3 + [iv] * 3 + [fv] * 3 + [rv] * 3
            + [pltpu.VMEM_SHARED((n_dst, D), jnp.float32)]
            + [sem] * 12
        ),
    )
    return fn(src_emb, idx_src, idx_dst, vals, zeros)


def _tc_layer(pu, pit, ps, cur_u, cur_i, cur_s, W_ui_i, b_ui_i, W_s_i, b_s_i,
              g0, bb0, g1, bb1):
    R = 1000
    grid = N_USERS // R
    eps = 1e-5

    def ln(x, g, b):
        m = jnp.mean(x, axis=-1, keepdims=True)
        xc = x - m
        v = jnp.mean(xc * xc, axis=-1, keepdims=True)
        return xc * lax.rsqrt(v + eps) * g + b

    def body(pu0, pu1, pit0, pit1, ps0, ps1, cu, ci, cs,
             wui, bui, ws, bs, g0r, b0r, g1r, b1r, ou, oi, osr):
        dn = (((1,), (1,)), ((), ()))
        au = pu0[...] + pu1[...]
        hu = lax.dot_general(au, wui[...], dn,
                             preferred_element_type=jnp.float32) + bui[...]
        ou[...] = ln(cu[...] + hu, g0r[...], b0r[...])
        ai = pit0[...] + pit1[...]
        hi = lax.dot_general(ai, wui[...], dn,
                             preferred_element_type=jnp.float32) + bui[...]
        oi[...] = ln(ci[...] + hi, g1r[...], b1r[...])
        as_ = ps0[...] + ps1[...]
        hs = lax.dot_general(as_, ws[...], dn,
                             preferred_element_type=jnp.float32) + bs[...]
        osr[...] = ln(cs[...] + hs, g0r[...], b0r[...])

    blk = pl.BlockSpec((R, D), lambda i: (i, 0))
    wblk = pl.BlockSpec((D, D), lambda i: (0, 0))
    vblk = pl.BlockSpec((1, D), lambda i: (0, 0))
    out_shape = jax.ShapeDtypeStruct((N_USERS, D), jnp.float32)
    return pl.pallas_call(
        body,
        grid=(grid,),
        in_specs=[blk] * 9 + [wblk, vblk, wblk, vblk, vblk, vblk, vblk, vblk],
        out_specs=[blk, blk, blk],
        out_shape=[out_shape, out_shape, out_shape],
    )(pu[0], pu[1], pit[0], pit[1], ps[0], ps[1], cur_u, cur_i, cur_s,
      W_ui_i, b_ui_i.reshape(1, D), W_s_i, b_s_i.reshape(1, D),
      g0.reshape(1, D), bb0.reshape(1, D), g1.reshape(1, D), bb1.reshape(1, D))


def kernel(user_emb, item_emb, ui_values, social_values, W_ui, b_ui, W_s, b_s,
           ln_g, ln_b, ui_edge_index, social_edge_index):
    row = ui_edge_index[0].astype(jnp.int32)
    col = ui_edge_index[1].astype(jnp.int32)
    srow = social_edge_index[0].astype(jnp.int32)
    scol = social_edge_index[1].astype(jnp.int32)

    row_p, col_p, uvals_p, e_ui = _pad_edges(row, col, ui_values)
    srow_p, scol_p, svals_p, e_soc = _pad_edges(srow, scol, social_values)

    zeros = jnp.zeros((N_PAD, D), jnp.float32)

    cur_u, cur_i, cur_s = user_emb, item_emb, user_emb
    ui_list = [user_emb]
    s_list = [user_emb]
    for i in range(N_LAYERS):
        pu = _seg_sum_partials(cur_i, col_p, row_p, uvals_p, zeros,
                               ep=e_ui, n_dst=N_PAD)
        pit = _seg_sum_partials(cur_u, row_p, col_p, uvals_p, zeros,
                                ep=e_ui, n_dst=N_PAD)
        ps = _seg_sum_partials(cur_s, scol_p, srow_p, svals_p, zeros,
                               ep=e_soc, n_dst=N_PAD)
        cur_u, cur_i, cur_s = _tc_layer(
            pu, pit, ps, cur_u, cur_i, cur_s,
            W_ui[i], b_ui[i], W_s[i], b_s[i],
            ln_g[2 * i], ln_b[2 * i], ln_g[2 * i + 1], ln_b[2 * i + 1])
        ui_list.append(cur_u)
        s_list.append(cur_s)

    return jnp.concatenate([jnp.stack(ui_list), jnp.stack(s_list),
                            cur_i[None]], axis=0)
